# Initial kernel scaffold; baseline (speedup 1.0000x reference)
import jax
import jax.numpy as jnp
from jax.experimental import pallas as pl


def kernel(x, edge_index, W1l, b1, W1r, W2l, b2, W2r, Wc, bc):
    raise NotImplementedError("write your pallas kernel here")


# speedup vs baseline: 3.3115x; 3.3115x over previous
import functools

import jax
import jax.numpy as jnp
from jax import lax
from jax.experimental import pallas as pl
from jax.experimental.pallas import tpu as pltpu
from jax.experimental.pallas import tpu_sc as plsc

_N = 10000
_E = 320000
_D = 128
_H = 1024
_C = 153

_NSC = 2
_NTILE = 16
_NP = 10240
_HALF = _NP // _NSC

_EPT = _E // _NTILE
_ECH = 2000


def _make_segsum(cm):
    fe = 128 // cm
    r = 10240 // cm
    npass = cm // 2
    app = 2 * fe + 32
    trash = 2 * fe + 16
    slabu = 10496
    zsh = slabu // _NTILE
    wsh = r * cm // _NTILE
    mesh = plsc.VectorSubcoreMesh(core_axis_name="c", subcore_axis_name="s")

    @functools.partial(
        pl.kernel,
        out_type=jax.ShapeDtypeStruct((_NP * cm, 128), jnp.float32),
        mesh=mesh,
        scratch_types=[
            pltpu.VMEM((_ECH,), jnp.int32),
            pltpu.VMEM((_ECH,), jnp.int32),
            pltpu.VMEM((app,), jnp.int32),
            pltpu.VMEM((app,), jnp.int32),
            pltpu.VMEM((128,), jnp.int32),
            pltpu.VMEM((128,), jnp.int32),
            pltpu.VMEM((128, 128), jnp.float32),
            pltpu.VMEM((16, 128), jnp.float32),
            pltpu.VMEM_SHARED((slabu, 128), jnp.float32),
            pltpu.SemaphoreType.DMA,
        ],
        compiler_params=pltpu.CompilerParams(needs_layout_passes=False),
    )
    def segsum(table, srcv, dstv, out, src_c, dst_c, psrc, pdst, fsrc, fdst,
               stage, zbuf, slab, sem):
        cid = lax.axis_index("c")
        sid = lax.axis_index("s")
        ebase = sid * _EPT
        lanes = lax.iota(jnp.int32, 16)

        def _zrow(rr, _):
            def _zcol(cc, _):
                zbuf[rr, pl.ds(cc * 16, 16)] = jnp.zeros((16,), jnp.float32)
                return 0
            lax.fori_loop(0, 8, _zcol, 0)
            return 0
        lax.fori_loop(0, 16, _zrow, 0)

        def fire_now():
            for j in range(fe // 16):
                sv = psrc[pl.ds(j * 16, 16)]
                dv = pdst[pl.ds(j * 16, 16)]
                for k in range(cm):
                    pos = lanes * cm + (j * 16 * cm + k)
                    plsc.store_scatter(fsrc, [pos], sv * cm + k)
                    plsc.store_scatter(fdst, [pos], dv * cm + k)
            pltpu.async_copy(table.at[fsrc], stage, sem).wait()
            pltpu.sync_copy(stage, slab.at[fdst], add=True)

        def pass_body(p, _):
            wbase = cid * _HALF + p * r
            def _z16(k, _):
                pltpu.sync_copy(zbuf, slab.at[pl.ds(sid * zsh + k * 16, 16)])
                return 0
            lax.fori_loop(0, zsh // 16, _z16, 0)
            plsc.subcore_barrier()

            def chunk_body(jc, nf):
                pltpu.sync_copy(srcv.at[pl.ds(ebase + jc * _ECH, _ECH)], src_c)
                pltpu.sync_copy(dstv.at[pl.ds(ebase + jc * _ECH, _ECH)], dst_c)

                def vec_body(jv, nf):
                    s16 = src_c[pl.ds(jv * 16, 16)]
                    d16 = dst_c[pl.ds(jv * 16, 16)]
                    dloc = d16 - wbase
                    m = (dloc >= 0) & (dloc < r)
                    csum = jnp.cumsum(jnp.where(m, 1, 0))
                    pos = jnp.where(m, nf + csum - 1, trash)
                    plsc.store_scatter(psrc, [pos], s16)
                    plsc.store_scatter(pdst, [pos], dloc)
                    nf2 = nf + jnp.max(csum)

                    def do_fire(v):
                        fire_now()
                        psrc[pl.ds(0, 16)] = psrc[pl.ds(fe, 16)]
                        pdst[pl.ds(0, 16)] = pdst[pl.ds(fe, 16)]
                        return v - fe

                    return lax.cond(nf2 >= fe, do_fire, lambda v: v, nf2)

                return lax.fori_loop(0, _ECH // 16, vec_body, nf)

            nf = lax.fori_loop(0, _EPT // _ECH, chunk_body, 0)

            for kk in range(fe // 16):
                psrc[pl.ds(nf + kk * 16, 16)] = jnp.zeros((16,), jnp.int32)
                pdst[pl.ds(nf + kk * 16, 16)] = jnp.full((16,), r, jnp.int32)
            fire_now()
            plsc.subcore_barrier()

            pltpu.sync_copy(slab.at[pl.ds(sid * wsh, wsh)],
                            out.at[pl.ds(wbase * cm + sid * wsh, wsh)])
            plsc.subcore_barrier()
            return 0

        lax.fori_loop(0, npass, pass_body, 0)

    return segsum


_segsum_l1 = _make_segsum(cm=2)
_segsum_l2 = _make_segsum(cm=8)

_ROWS_BLK = 400
_GRID = _N // _ROWS_BLK


def _elu(z):
    return jnp.where(z > 0, z, jnp.exp(jnp.minimum(z, 0.0)) - 1.0)


def _tc1_body(s_ref, x_ref, wl_ref, b_ref, wr_ref, h_ref):
    s = s_ref[...]
    rcp = 1.0 / jnp.maximum(s[:, 128:129], 1.0)
    mean = s[:, :128] * rcp
    z = (jnp.dot(mean, wl_ref[...], preferred_element_type=jnp.float32)
         + b_ref[...]
         + jnp.dot(x_ref[...], wr_ref[...], preferred_element_type=jnp.float32))
    h_ref[...] = _elu(z)


def _tc1(sums1, x, W1l, b1, W1r):
    return pl.pallas_call(
        _tc1_body,
        grid=(_GRID,),
        in_specs=[
            pl.BlockSpec((_ROWS_BLK, 256), lambda i: (i, 0)),
            pl.BlockSpec((_ROWS_BLK, _D), lambda i: (i, 0)),
            pl.BlockSpec((_D, _H), lambda i: (0, 0)),
            pl.BlockSpec((1, _H), lambda i: (0, 0)),
            pl.BlockSpec((_D, _H), lambda i: (0, 0)),
        ],
        out_specs=pl.BlockSpec((_ROWS_BLK, _H), lambda i: (i, 0)),
        out_shape=jax.ShapeDtypeStruct((_N, _H), jnp.float32),
    )(sums1, x, W1l, b1, W1r)


def _tc2_body(s2_ref, s1_ref, h_ref, wl_ref, b_ref, wr_ref, wc_ref, bc_ref,
              o_ref):
    rcp = 1.0 / jnp.maximum(s1_ref[:, 128:129], 1.0)
    mean = s2_ref[...] * rcp
    z = (jnp.dot(mean, wl_ref[...], preferred_element_type=jnp.float32)
         + b_ref[...]
         + jnp.dot(h_ref[...], wr_ref[...], preferred_element_type=jnp.float32))
    h2 = _elu(z)
    o_ref[...] = jnp.dot(h2, wc_ref[...], preferred_element_type=jnp.float32) + bc_ref[...]


def _tc2(sums2, sums1, h, W2l, b2, W2r, Wcp, bcp):
    return pl.pallas_call(
        _tc2_body,
        grid=(_GRID,),
        in_specs=[
            pl.BlockSpec((_ROWS_BLK, _H), lambda i: (i, 0)),
            pl.BlockSpec((_ROWS_BLK, 256), lambda i: (i, 0)),
            pl.BlockSpec((_ROWS_BLK, _H), lambda i: (i, 0)),
            pl.BlockSpec((_H, _H), lambda i: (0, 0)),
            pl.BlockSpec((1, _H), lambda i: (0, 0)),
            pl.BlockSpec((_H, _H), lambda i: (0, 0)),
            pl.BlockSpec((_H, 256), lambda i: (0, 0)),
            pl.BlockSpec((1, 256), lambda i: (0, 0)),
        ],
        out_specs=pl.BlockSpec((_ROWS_BLK, 256), lambda i: (i, 0)),
        out_shape=jax.ShapeDtypeStruct((_N, 256), jnp.float32),
    )(sums2, sums1, h, W2l, b2, W2r, Wcp, bcp)


def kernel(x, edge_index, W1l, b1, W1r, W2l, b2, W2r, Wc, bc):
    src = edge_index[0].astype(jnp.int32)
    dst = edge_index[1].astype(jnp.int32)

    x_aug = jnp.concatenate(
        [x, jnp.ones((_N, 1), jnp.float32), jnp.zeros((_N, 127), jnp.float32)],
        axis=1)

    sums1 = _segsum_l1(x_aug.reshape(_N * 2, 128), src, dst)
    sums1 = sums1.reshape(_NP, 256)[:_N]
    h = _tc1(sums1, x, W1l, b1.reshape(1, _H), W1r)
    sums2 = _segsum_l2(h.reshape(_N * 8, 128), src, dst)
    sums2 = sums2.reshape(_NP, _H)[:_N]
    Wcp = jnp.pad(Wc, ((0, 0), (0, 256 - _C)))
    bcp = jnp.pad(bc, (0, 256 - _C)).reshape(1, 256)
    out = _tc2(sums2, sums1, h, W2l, b2.reshape(1, _H), W2r, Wcp, bcp)
    return out[:, :_C]
